# baseline (speedup 1.0000x reference)
import functools

import jax
import jax.numpy as jnp
from jax.experimental import pallas as pl
from jax.experimental.pallas import tpu as pltpu

B, C, L = 32, 4, 2048
HID, ZD, K = 256, 64, 1024
BETA = 0.25
N = B * L

TILE = 1024
HALF = TILE // 2
NSTEPS = N // TILE
LOSS_SCALE = (1.0 + BETA) / (N * ZD)

_INV_SQRT2 = 0.7071067811865476


def _gelu(x):
    return x * (0.5 * (1.0 + jax.lax.erf(x * _INV_SQRT2)))


def _bdot(a, b):
    return jnp.dot(a, b, preferred_element_type=jnp.float32)


def _bf(x):
    return x.astype(jnp.bfloat16)


def _vqvae_body(xt_ref, w1_ref, b1_ref, w2_ref, b2_ref, w3_ref, b3_ref,
                cbt2_ref, cb_ref, cbf_ref, dw1_ref, db1_ref, dw2_ref, db2_ref,
                dw3_ref, db3_ref, out_ref, loss_ref, cnorm_ref):
    i = pl.program_id(0)

    @pl.when(i == 0)
    def _init():
        loss_ref[...] = jnp.zeros((1, 1), jnp.float32)
        cbf = cbf_ref[...]
        cnorm_ref[...] = jnp.sum(cbf * cbf, axis=1)[None, :]

    def half(sl):
        h = _gelu(_bdot(xt_ref[sl], w1_ref[...]) + b1_ref[...])
        h = _gelu(_bdot(_bf(h), w2_ref[...]) + b2_ref[...])
        z = _bdot(_bf(h), w3_ref[...]) + b3_ref[...]

        znorm = jnp.sum(z * z, axis=1, keepdims=True)
        d = (znorm + _bdot(_bf(z), cbt2_ref[...])) + cnorm_ref[...]
        dmin = jnp.min(d, axis=1, keepdims=True)
        oh = (d == dmin).astype(jnp.bfloat16)
        zq = _bdot(oh, cb_ref[...])

        diff = z - zq
        part = jnp.sum(diff * diff).reshape(1, 1)

        g = _gelu(_bdot(_bf(zq), dw1_ref[...]) + db1_ref[...])
        g = _gelu(_bdot(_bf(g), dw2_ref[...]) + db2_ref[...])
        out_ref[sl] = _bdot(_bf(g), dw3_ref[...]) + db3_ref[...]
        return part

    pa = half(pl.ds(0, HALF))
    pb = half(pl.ds(HALF, HALF))
    loss_ref[...] += pa + pb

    @pl.when(i == NSTEPS - 1)
    def _final():
        loss_ref[...] = loss_ref[...] * LOSS_SCALE


@functools.partial(jax.jit, static_argnames=())
def kernel(x, mean, std, enc_w1, enc_b1, enc_w2, enc_b2, enc_w3, enc_b3,
           codebook, dec_w1, dec_b1, dec_w2, dec_b2, dec_w3, dec_b3):
    f32 = jnp.float32
    bf16 = jnp.bfloat16
    m = mean.reshape(C)
    s = std.reshape(C)
    w1f = (enc_w1 / s[:, None]).astype(bf16)
    b1f = (enc_b1 - (m / s) @ enc_w1)[None, :]
    w3f = (dec_w3 * s[None, :]).astype(bf16)
    b3f = (dec_b3 * s + m)[None, :]

    xt = jnp.transpose(x, (0, 2, 1)).reshape(N, C).astype(bf16)

    full = lambda shape: pl.BlockSpec(shape, lambda i: (0, 0))
    rec_flat, loss = pl.pallas_call(
        _vqvae_body,
        grid=(NSTEPS,),
        in_specs=[
            pl.BlockSpec((TILE, C), lambda i: (i, 0)),
            full((C, HID)), full((1, HID)),
            full((HID, HID)), full((1, HID)),
            full((HID, ZD)), full((1, ZD)),
            full((ZD, K)),
            full((K, ZD)),
            full((K, ZD)),
            full((ZD, HID)), full((1, HID)),
            full((HID, HID)), full((1, HID)),
            full((HID, C)), full((1, C)),
        ],
        out_specs=[
            pl.BlockSpec((TILE, C), lambda i: (i, 0)),
            pl.BlockSpec((1, 1), lambda i: (0, 0)),
        ],
        out_shape=[
            jax.ShapeDtypeStruct((N, C), f32),
            jax.ShapeDtypeStruct((1, 1), f32),
        ],
        scratch_shapes=[pltpu.VMEM((1, K), f32)],
    )(xt, w1f, b1f, enc_w2.astype(bf16), enc_b2[None, :],
      enc_w3.astype(bf16), enc_b3[None, :],
      (codebook.T * -2.0).astype(bf16), codebook.astype(bf16), codebook,
      dec_w1.astype(bf16), dec_b1[None, :], dec_w2.astype(bf16),
      dec_b2[None, :], w3f, b3f)

    rec = jnp.transpose(rec_flat.reshape(B, L, C), (0, 2, 1))
    return rec, loss.reshape(())


# speedup vs baseline: 1.1530x; 1.1530x over previous
import functools

import jax
import jax.numpy as jnp
from jax.experimental import pallas as pl
from jax.experimental.pallas import tpu as pltpu

B, C, L = 32, 4, 2048
HID, ZD, K = 256, 64, 1024
BETA = 0.25
N = B * L

TILE = 1024
HALF = TILE // 2
NSTEPS = N // TILE
LOSS_SCALE = (1.0 + BETA) / (N * ZD)

_INV_SQRT2 = 0.7071067811865476


def _gelu(x):
    return x * (0.5 * (1.0 + jax.lax.erf(x * _INV_SQRT2)))


def _bdot(a, b):
    return jnp.dot(a, b, preferred_element_type=jnp.float32)


def _bf(x):
    return x.astype(jnp.bfloat16)


def _vqvae_body(xt_ref, w1_ref, b1_ref, w2_ref, b2_ref, w3_ref, b3_ref,
                cbt2_ref, cb_ref, cbf_ref, dw1_ref, db1_ref, dw2_ref, db2_ref,
                dw3_ref, db3_ref, out_ref, loss_ref, cnorm_ref):
    i = pl.program_id(0)

    @pl.when(i == 0)
    def _init():
        loss_ref[...] = jnp.zeros((1, 1), jnp.float32)
        cbf = cbf_ref[...]
        cnorm_ref[...] = jnp.sum(cbf * cbf, axis=1)[None, :]

    def half(sl):
        h = _gelu(_bdot(xt_ref[sl], w1_ref[...]) + b1_ref[...])
        h = _gelu(_bdot(_bf(h), w2_ref[...]) + b2_ref[...])
        z = _bdot(_bf(h), w3_ref[...]) + b3_ref[...]

        znorm = jnp.sum(z * z, axis=1, keepdims=True)
        d = (znorm + _bdot(_bf(z), cbt2_ref[...])) + cnorm_ref[...]
        dmin = jnp.min(d, axis=1, keepdims=True)
        oh = (d == dmin).astype(jnp.bfloat16)
        zq = _bdot(oh, cb_ref[...])

        diff = z - zq
        part = jnp.sum(diff * diff).reshape(1, 1)

        g = _gelu(_bdot(_bf(zq), dw1_ref[...]) + db1_ref[...])
        g = _gelu(_bdot(_bf(g), dw2_ref[...]) + db2_ref[...])
        out_ref[sl] = _bdot(_bf(g), dw3_ref[...]) + db3_ref[...]
        return part

    loss_ref[...] += half(pl.ds(0, TILE))

    @pl.when(i == NSTEPS - 1)
    def _final():
        loss_ref[...] = loss_ref[...] * LOSS_SCALE


@functools.partial(jax.jit, static_argnames=())
def kernel(x, mean, std, enc_w1, enc_b1, enc_w2, enc_b2, enc_w3, enc_b3,
           codebook, dec_w1, dec_b1, dec_w2, dec_b2, dec_w3, dec_b3):
    f32 = jnp.float32
    bf16 = jnp.bfloat16
    m = mean.reshape(C)
    s = std.reshape(C)
    w1f = (enc_w1 / s[:, None]).astype(bf16)
    b1f = (enc_b1 - (m / s) @ enc_w1)[None, :]
    w3f = (dec_w3 * s[None, :]).astype(bf16)
    b3f = (dec_b3 * s + m)[None, :]

    xt = jnp.transpose(x, (0, 2, 1)).reshape(N, C).astype(bf16)

    full = lambda shape: pl.BlockSpec(shape, lambda i: (0, 0))
    rec_flat, loss = pl.pallas_call(
        _vqvae_body,
        grid=(NSTEPS,),
        in_specs=[
            pl.BlockSpec((TILE, C), lambda i: (i, 0)),
            full((C, HID)), full((1, HID)),
            full((HID, HID)), full((1, HID)),
            full((HID, ZD)), full((1, ZD)),
            full((ZD, K)),
            full((K, ZD)),
            full((K, ZD)),
            full((ZD, HID)), full((1, HID)),
            full((HID, HID)), full((1, HID)),
            full((HID, C)), full((1, C)),
        ],
        out_specs=[
            pl.BlockSpec((TILE, C), lambda i: (i, 0)),
            pl.BlockSpec((1, 1), lambda i: (0, 0)),
        ],
        out_shape=[
            jax.ShapeDtypeStruct((N, C), f32),
            jax.ShapeDtypeStruct((1, 1), f32),
        ],
        scratch_shapes=[pltpu.VMEM((1, K), f32)],
    )(xt, w1f, b1f, enc_w2.astype(bf16), enc_b2[None, :],
      enc_w3.astype(bf16), enc_b3[None, :],
      (codebook.T * -2.0).astype(bf16), codebook.astype(bf16), codebook,
      dec_w1.astype(bf16), dec_b1[None, :], dec_w2.astype(bf16),
      dec_b2[None, :], w3f, b3f)

    rec = jnp.transpose(rec_flat.reshape(B, L, C), (0, 2, 1))
    return rec, loss.reshape(())


# speedup vs baseline: 1.2205x; 1.0585x over previous
import functools

import jax
import jax.numpy as jnp
from jax.experimental import pallas as pl
from jax.experimental.pallas import tpu as pltpu

B, C, L = 32, 4, 2048
HID, ZD, K = 256, 64, 1024
BETA = 0.25
N = B * L

TILE = 2048
HALF = TILE // 2
NSTEPS = N // TILE
LOSS_SCALE = (1.0 + BETA) / (N * ZD)

_INV_SQRT2 = 0.7071067811865476


def _gelu(x):
    return x * (0.5 * (1.0 + jax.lax.erf(x * _INV_SQRT2)))


def _bdot(a, b):
    return jnp.dot(a, b, preferred_element_type=jnp.float32)


def _bf(x):
    return x.astype(jnp.bfloat16)


def _vqvae_body(xt_ref, w1_ref, b1_ref, w2_ref, b2_ref, w3_ref, b3_ref,
                cbt2_ref, cb_ref, cbf_ref, dw1_ref, db1_ref, dw2_ref, db2_ref,
                dw3_ref, db3_ref, out_ref, loss_ref, cnorm_ref):
    i = pl.program_id(0)

    @pl.when(i == 0)
    def _init():
        loss_ref[...] = jnp.zeros((1, 1), jnp.float32)
        cbf = cbf_ref[...]
        cnorm_ref[...] = jnp.sum(cbf * cbf, axis=1)[None, :]

    def half(sl):
        h = _gelu(_bdot(xt_ref[sl], w1_ref[...]) + b1_ref[...])
        h = _gelu(_bdot(_bf(h), w2_ref[...]) + b2_ref[...])
        z = _bdot(_bf(h), w3_ref[...]) + b3_ref[...]

        znorm = jnp.sum(z * z, axis=1, keepdims=True)
        d = (znorm + _bdot(_bf(z), cbt2_ref[...])) + cnorm_ref[...]
        dmin = jnp.min(d, axis=1, keepdims=True)
        oh = (d == dmin).astype(jnp.bfloat16)
        zq = _bdot(oh, cb_ref[...])

        diff = z - zq
        part = jnp.sum(diff * diff).reshape(1, 1)

        g = _gelu(_bdot(_bf(zq), dw1_ref[...]) + db1_ref[...])
        g = _gelu(_bdot(_bf(g), dw2_ref[...]) + db2_ref[...])
        out_ref[sl] = _bdot(_bf(g), dw3_ref[...]) + db3_ref[...]
        return part

    loss_ref[...] += half(pl.ds(0, TILE))

    @pl.when(i == NSTEPS - 1)
    def _final():
        loss_ref[...] = loss_ref[...] * LOSS_SCALE


@functools.partial(jax.jit, static_argnames=())
def kernel(x, mean, std, enc_w1, enc_b1, enc_w2, enc_b2, enc_w3, enc_b3,
           codebook, dec_w1, dec_b1, dec_w2, dec_b2, dec_w3, dec_b3):
    f32 = jnp.float32
    bf16 = jnp.bfloat16
    m = mean.reshape(C)
    s = std.reshape(C)
    w1f = (enc_w1 / s[:, None]).astype(bf16)
    b1f = (enc_b1 - (m / s) @ enc_w1)[None, :]
    w3f = (dec_w3 * s[None, :]).astype(bf16)
    b3f = (dec_b3 * s + m)[None, :]

    xt = jnp.transpose(x, (0, 2, 1)).reshape(N, C).astype(bf16)

    full = lambda shape: pl.BlockSpec(shape, lambda i: (0, 0))
    rec_flat, loss = pl.pallas_call(
        _vqvae_body,
        grid=(NSTEPS,),
        in_specs=[
            pl.BlockSpec((TILE, C), lambda i: (i, 0)),
            full((C, HID)), full((1, HID)),
            full((HID, HID)), full((1, HID)),
            full((HID, ZD)), full((1, ZD)),
            full((ZD, K)),
            full((K, ZD)),
            full((K, ZD)),
            full((ZD, HID)), full((1, HID)),
            full((HID, HID)), full((1, HID)),
            full((HID, C)), full((1, C)),
        ],
        out_specs=[
            pl.BlockSpec((TILE, C), lambda i: (i, 0)),
            pl.BlockSpec((1, 1), lambda i: (0, 0)),
        ],
        out_shape=[
            jax.ShapeDtypeStruct((N, C), f32),
            jax.ShapeDtypeStruct((1, 1), f32),
        ],
        scratch_shapes=[pltpu.VMEM((1, K), f32)],
    )(xt, w1f, b1f, enc_w2.astype(bf16), enc_b2[None, :],
      enc_w3.astype(bf16), enc_b3[None, :],
      (codebook.T * -2.0).astype(bf16), codebook.astype(bf16), codebook,
      dec_w1.astype(bf16), dec_b1[None, :], dec_w2.astype(bf16),
      dec_b2[None, :], w3f, b3f)

    rec = jnp.transpose(rec_flat.reshape(B, L, C), (0, 2, 1))
    return rec, loss.reshape(())


# speedup vs baseline: 1.2572x; 1.0301x over previous
import functools

import jax
import jax.numpy as jnp
from jax.experimental import pallas as pl
from jax.experimental.pallas import tpu as pltpu

B, C, L = 32, 4, 2048
HID, ZD, K = 256, 64, 1024
BETA = 0.25
N = B * L

TILE = 4096
HALF = TILE // 2
NSTEPS = N // TILE
LOSS_SCALE = (1.0 + BETA) / (N * ZD)

_INV_SQRT2 = 0.7071067811865476


def _gelu(x):
    return x * (0.5 * (1.0 + jax.lax.erf(x * _INV_SQRT2)))


def _bdot(a, b):
    return jnp.dot(a, b, preferred_element_type=jnp.float32)


def _bf(x):
    return x.astype(jnp.bfloat16)


def _vqvae_body(xt_ref, w1_ref, b1_ref, w2_ref, b2_ref, w3_ref, b3_ref,
                cbt2_ref, cb_ref, cbf_ref, dw1_ref, db1_ref, dw2_ref, db2_ref,
                dw3_ref, db3_ref, out_ref, loss_ref, cnorm_ref):
    i = pl.program_id(0)

    @pl.when(i == 0)
    def _init():
        loss_ref[...] = jnp.zeros((1, 1), jnp.float32)
        cbf = cbf_ref[...]
        cnorm_ref[...] = jnp.sum(cbf * cbf, axis=1)[None, :]

    def half(sl):
        h = _gelu(_bdot(xt_ref[sl], w1_ref[...]) + b1_ref[...])
        h = _gelu(_bdot(_bf(h), w2_ref[...]) + b2_ref[...])
        z = _bdot(_bf(h), w3_ref[...]) + b3_ref[...]

        znorm = jnp.sum(z * z, axis=1, keepdims=True)
        d = (znorm + _bdot(_bf(z), cbt2_ref[...])) + cnorm_ref[...]
        dmin = jnp.min(d, axis=1, keepdims=True)
        oh = (d == dmin).astype(jnp.bfloat16)
        zq = _bdot(oh, cb_ref[...])

        diff = z - zq
        part = jnp.sum(diff * diff).reshape(1, 1)

        g = _gelu(_bdot(_bf(zq), dw1_ref[...]) + db1_ref[...])
        g = _gelu(_bdot(_bf(g), dw2_ref[...]) + db2_ref[...])
        out_ref[sl] = _bdot(_bf(g), dw3_ref[...]) + db3_ref[...]
        return part

    loss_ref[...] += half(pl.ds(0, TILE))

    @pl.when(i == NSTEPS - 1)
    def _final():
        loss_ref[...] = loss_ref[...] * LOSS_SCALE


@functools.partial(jax.jit, static_argnames=())
def kernel(x, mean, std, enc_w1, enc_b1, enc_w2, enc_b2, enc_w3, enc_b3,
           codebook, dec_w1, dec_b1, dec_w2, dec_b2, dec_w3, dec_b3):
    f32 = jnp.float32
    bf16 = jnp.bfloat16
    m = mean.reshape(C)
    s = std.reshape(C)
    w1f = (enc_w1 / s[:, None]).astype(bf16)
    b1f = (enc_b1 - (m / s) @ enc_w1)[None, :]
    w3f = (dec_w3 * s[None, :]).astype(bf16)
    b3f = (dec_b3 * s + m)[None, :]

    xt = jnp.transpose(x, (0, 2, 1)).reshape(N, C).astype(bf16)

    full = lambda shape: pl.BlockSpec(shape, lambda i: (0, 0))
    rec_flat, loss = pl.pallas_call(
        _vqvae_body,
        grid=(NSTEPS,),
        in_specs=[
            pl.BlockSpec((TILE, C), lambda i: (i, 0)),
            full((C, HID)), full((1, HID)),
            full((HID, HID)), full((1, HID)),
            full((HID, ZD)), full((1, ZD)),
            full((ZD, K)),
            full((K, ZD)),
            full((K, ZD)),
            full((ZD, HID)), full((1, HID)),
            full((HID, HID)), full((1, HID)),
            full((HID, C)), full((1, C)),
        ],
        out_specs=[
            pl.BlockSpec((TILE, C), lambda i: (i, 0)),
            pl.BlockSpec((1, 1), lambda i: (0, 0)),
        ],
        out_shape=[
            jax.ShapeDtypeStruct((N, C), f32),
            jax.ShapeDtypeStruct((1, 1), f32),
        ],
        scratch_shapes=[pltpu.VMEM((1, K), f32)],
    )(xt, w1f, b1f, enc_w2.astype(bf16), enc_b2[None, :],
      enc_w3.astype(bf16), enc_b3[None, :],
      (codebook.T * -2.0).astype(bf16), codebook.astype(bf16), codebook,
      dec_w1.astype(bf16), dec_b1[None, :], dec_w2.astype(bf16),
      dec_b2[None, :], w3f, b3f)

    rec = jnp.transpose(rec_flat.reshape(B, L, C), (0, 2, 1))
    return rec, loss.reshape(())


# speedup vs baseline: 1.4301x; 1.1375x over previous
import functools

import jax
import jax.numpy as jnp
from jax.experimental import pallas as pl
from jax.experimental.pallas import tpu as pltpu

B, C, L = 32, 4, 2048
HID, ZD, K = 256, 64, 1024
BETA = 0.25
N = B * L

BSTEP = 2
NSTEPS = B // BSTEP
LOSS_SCALE = (1.0 + BETA) / (N * ZD)

_INV_SQRT2 = 0.7071067811865476


def _gelu(x):
    return x * (0.5 * (1.0 + jax.lax.erf(x * _INV_SQRT2)))


def _bdot(a, b):
    return jnp.dot(a, b, preferred_element_type=jnp.float32)


def _bf(x):
    return x.astype(jnp.bfloat16)


def _vqvae_body(x_ref, w1_ref, b1_ref, w2_ref, b2_ref, w3_ref, b3_ref,
                cbt2_ref, cb_ref, cbf_ref, dw1_ref, db1_ref, dw2_ref, db2_ref,
                dw3_ref, db3_ref, out_ref, loss_ref, cnorm_ref):
    i = pl.program_id(0)

    @pl.when(i == 0)
    def _init():
        loss_ref[...] = jnp.zeros((1, 1), jnp.float32)
        cbf = cbf_ref[...]
        cnorm_ref[...] = jnp.sum(cbf * cbf, axis=1)[None, :]

    part = jnp.zeros((1, 1), jnp.float32)
    for b in range(BSTEP):
        xb = _bf(x_ref[b])
        h = jax.lax.dot_general(
            xb, w1_ref[...], (((0,), (0,)), ((), ())),
            preferred_element_type=jnp.float32)
        h = _gelu(h + b1_ref[...])
        h = _gelu(_bdot(_bf(h), w2_ref[...]) + b2_ref[...])
        z = _bdot(_bf(h), w3_ref[...]) + b3_ref[...]

        znorm = jnp.sum(z * z, axis=1, keepdims=True)
        d = (znorm + _bdot(_bf(z), cbt2_ref[...])) + cnorm_ref[...]
        dmin = jnp.min(d, axis=1, keepdims=True)
        oh = (d == dmin).astype(jnp.bfloat16)
        zq = _bdot(oh, cb_ref[...])

        diff = z - zq
        part = part + jnp.sum(diff * diff).reshape(1, 1)

        g = _gelu(_bdot(_bf(zq), dw1_ref[...]) + db1_ref[...])
        g = _gelu(_bdot(_bf(g), dw2_ref[...]) + db2_ref[...])
        outb = jax.lax.dot_general(
            dw3_ref[...], _bf(g), (((0,), (1,)), ((), ())),
            preferred_element_type=jnp.float32)
        out_ref[b] = outb + db3_ref[...]

    loss_ref[...] += part

    @pl.when(i == NSTEPS - 1)
    def _final():
        loss_ref[...] = loss_ref[...] * LOSS_SCALE


@functools.partial(jax.jit, static_argnames=())
def kernel(x, mean, std, enc_w1, enc_b1, enc_w2, enc_b2, enc_w3, enc_b3,
           codebook, dec_w1, dec_b1, dec_w2, dec_b2, dec_w3, dec_b3):
    f32 = jnp.float32
    bf16 = jnp.bfloat16
    m = mean.reshape(C)
    s = std.reshape(C)
    w1f = (enc_w1 / s[:, None]).astype(bf16)
    b1f = (enc_b1 - (m / s) @ enc_w1)[None, :]
    w3f = (dec_w3 * s[None, :]).astype(bf16)
    b3f = (dec_b3 * s + m)[:, None]

    full = lambda shape: pl.BlockSpec(shape, lambda i: tuple(0 for _ in shape))
    rec, loss = pl.pallas_call(
        _vqvae_body,
        grid=(NSTEPS,),
        in_specs=[
            pl.BlockSpec((BSTEP, C, L), lambda i: (i, 0, 0)),
            full((C, HID)), full((1, HID)),
            full((HID, HID)), full((1, HID)),
            full((HID, ZD)), full((1, ZD)),
            full((ZD, K)),
            full((K, ZD)),
            full((K, ZD)),
            full((ZD, HID)), full((1, HID)),
            full((HID, HID)), full((1, HID)),
            full((HID, C)), full((C, 1)),
        ],
        out_specs=[
            pl.BlockSpec((BSTEP, C, L), lambda i: (i, 0, 0)),
            pl.BlockSpec((1, 1), lambda i: (0, 0)),
        ],
        out_shape=[
            jax.ShapeDtypeStruct((B, C, L), f32),
            jax.ShapeDtypeStruct((1, 1), f32),
        ],
        scratch_shapes=[pltpu.VMEM((1, K), f32)],
    )(x, w1f, b1f, enc_w2.astype(bf16), enc_b2[None, :],
      enc_w3.astype(bf16), enc_b3[None, :],
      (codebook.T * -2.0).astype(bf16), codebook.astype(bf16), codebook,
      dec_w1.astype(bf16), dec_b1[None, :], dec_w2.astype(bf16),
      dec_b2[None, :], w3f, b3f)

    return rec, loss.reshape(())
